# baseline (speedup 1.0000x reference)
import functools

import jax
import jax.numpy as jnp
from jax import lax
from jax.experimental import pallas as pl
from jax.experimental.pallas import tpu as pltpu
from jax.experimental.pallas import tpu_sc as plsc

B, C, H, W_IMG, K = 8, 768, 32, 32, 1024
HW = H * W_IMG
_NTILE = 16
CPT = C // _NTILE
_NCV = CPT // 16


def _norm_body(c_ref, o_ref):
    c = c_ref[...]
    n = jnp.sqrt(jnp.sum(c * c, axis=1, keepdims=True))
    o_ref[...] = (c / jnp.maximum(n, 1e-12)).astype(jnp.bfloat16)


def _assign_body(f_ref, cn_ref, idx_ref, cnt_ref):
    f = f_ref[0]
    n = jnp.sqrt(jnp.sum(f * f, axis=0, keepdims=True))
    fn = (f / jnp.maximum(n, 1e-12)).astype(jnp.bfloat16)
    cn = cn_ref[...]
    scores = lax.dot_general(cn, fn, (((1,), (0,)), ((), ())),
                             preferred_element_type=jnp.float32)
    m = jnp.max(scores, axis=0, keepdims=True)
    kio = lax.broadcasted_iota(jnp.int32, (K, HW), 0)
    idx = jnp.min(jnp.where(scores >= m, kio, K), axis=0)
    idx_ref[0, 0, :] = idx
    cnt_ref[0, 0, :] = jnp.sum((kio == idx[None, :]).astype(jnp.float32),
                               axis=1)


def _assign(f3, cn):
    return pl.pallas_call(
        _assign_body,
        grid=(B,),
        in_specs=[
            pl.BlockSpec((1, C, HW), lambda b: (b, 0, 0)),
            pl.BlockSpec((K, C), lambda b: (0, 0)),
        ],
        out_specs=[
            pl.BlockSpec((1, 1, HW), lambda b: (b, 0, 0)),
            pl.BlockSpec((1, 1, K), lambda b: (b, 0, 0)),
        ],
        out_shape=[
            jax.ShapeDtypeStruct((B, 1, HW), jnp.int32),
            jax.ShapeDtypeStruct((B, 1, K), jnp.float32),
        ],
    )(f3, cn)


def _sc_body(ft_hbm, idx_hbm, cnt_hbm, g_hbm, gc_hbm,
             ftile, acc, idxv, cntv, gcout):
    cid = lax.axis_index("c")
    sid = lax.axis_index("s")
    col0 = sid * CPT
    iota16 = lax.iota(jnp.int32, 16)
    rows = [jv * 16 + iota16 for jv in range(_NCV)]

    for bb in range(B // 2):
        b = cid * (B // 2) + bb
        prow = b * HW

        pltpu.sync_copy(idx_hbm.at[pl.ds(prow, HW)], idxv)
        pltpu.sync_copy(ft_hbm.at[pl.ds(b * C + col0, CPT)], ftile)

        def _zero(i, _):
            acc[i // (K // 16), pl.ds((i % (K // 16)) * 16, 16)] = (
                jnp.zeros((16,), jnp.float32))
            return 0
        lax.fori_loop(0, CPT * (K // 16), _zero, 0)

        @pl.when(sid == 0)
        def _():
            pltpu.sync_copy(cnt_hbm.at[pl.ds(b * K, K)], cntv)

            def _gcl(gi, _):
                i16 = idxv[pl.ds(gi * 16, 16)]
                gcout[pl.ds(gi * 16, 16)] = plsc.load_gather(cntv, [i16])
                return 0
            lax.fori_loop(0, HW // 16, _gcl, 0)
            pltpu.sync_copy(gcout, gc_hbm.at[pl.ds(prow, HW)])

        def _scat(gi, _):
            i16 = idxv[pl.ds(gi * 16, 16)]
            for j in range(16):
                s = jnp.sum(jnp.where(iota16 == j, i16, 0))
                p = gi * 16 + j
                pcol = jnp.zeros((16,), jnp.int32) + p
                scol = jnp.zeros((16,), jnp.int32) + s
                for jv in range(_NCV):
                    x16 = plsc.load_gather(ftile, [rows[jv], pcol])
                    plsc.addupdate_scatter(acc, [rows[jv], scol], x16)
            return 0
        lax.fori_loop(0, HW // 16, _scat, 0)

        def _gath(gi, _):
            i16 = idxv[pl.ds(gi * 16, 16)]
            for j in range(16):
                s = jnp.sum(jnp.where(iota16 == j, i16, 0))
                p = gi * 16 + j
                pcol = jnp.zeros((16,), jnp.int32) + p
                scol = jnp.zeros((16,), jnp.int32) + s
                for jv in range(_NCV):
                    g16 = plsc.load_gather(acc, [rows[jv], scol])
                    plsc.store_scatter(ftile, [rows[jv], pcol], g16)
            return 0
        lax.fori_loop(0, HW // 16, _gath, 0)
        pltpu.sync_copy(ftile, g_hbm.at[pl.ds(b * C + col0, CPT)])


def _sc_scatter_gather(ft_flat, idx_flat, cnt_flat):
    mesh = plsc.VectorSubcoreMesh(core_axis_name="c", subcore_axis_name="s")
    fn = functools.partial(
        pl.kernel,
        mesh=mesh,
        compiler_params=pltpu.CompilerParams(
            use_tc_tiling_on_sc=False, needs_layout_passes=False),
        out_type=[
            jax.ShapeDtypeStruct((B * C, HW), jnp.float32),
            jax.ShapeDtypeStruct((B * HW,), jnp.float32),
        ],
        scratch_types=[
            pltpu.VMEM((CPT, HW), jnp.float32),
            pltpu.VMEM((CPT, K), jnp.float32),
            pltpu.VMEM((HW,), jnp.int32),
            pltpu.VMEM((K,), jnp.float32),
            pltpu.VMEM((HW,), jnp.float32),
        ],
    )(_sc_body)
    return fn(ft_flat, idx_flat, cnt_flat)


def _finish_body(f_ref, g_ref, gc_ref, w_ref, b_ref, out_ref):
    f = f_ref[0]
    g = g_ref[0]
    cnt = gc_ref[0]
    cents = g / jnp.maximum(cnt, 1.0)
    delta = cents - f
    alpha = jnp.exp(-jnp.mean(delta * delta, axis=0, keepdims=True))
    x = f + alpha * delta
    out = lax.dot_general(w_ref[...], x, (((1,), (0,)), ((), ())),
                          preferred_element_type=jnp.float32)
    out_ref[0] = jnp.maximum(out + b_ref[...], 0.0)


def _finish(f3, g, gc, Wfc, b2):
    return pl.pallas_call(
        _finish_body,
        grid=(B,),
        in_specs=[
            pl.BlockSpec((1, C, HW), lambda b: (b, 0, 0)),
            pl.BlockSpec((1, C, HW), lambda b: (b, 0, 0)),
            pl.BlockSpec((1, 1, HW), lambda b: (b, 0, 0)),
            pl.BlockSpec((C, C), lambda b: (0, 0)),
            pl.BlockSpec((C, 1), lambda b: (0, 0)),
        ],
        out_specs=pl.BlockSpec((1, C, HW), lambda b: (b, 0, 0)),
        out_shape=jax.ShapeDtypeStruct((B, C, HW), jnp.float32),
    )(f3, g, gc, Wfc, b2)


def kernel(features, centroids, Wfc, bfc):
    f3 = features.reshape(B, C, HW)
    cn = pl.pallas_call(
        _norm_body,
        out_shape=jax.ShapeDtypeStruct((K, C), jnp.bfloat16),
    )(centroids)
    idx3, cnt = _assign(f3, cn)
    g, gc = _sc_scatter_gather(features.reshape(B * C, HW),
                               idx3.reshape(B * HW), cnt.reshape(B * K))
    out = _finish(f3, g.reshape(B, C, HW), gc.reshape(B, 1, HW),
                  Wfc, bfc.reshape(C, 1))
    return out.reshape(B, C, H, W_IMG)


# speedup vs baseline: 1.6571x; 1.6571x over previous
import functools

import jax
import jax.numpy as jnp
from jax import lax
from jax.experimental import pallas as pl
from jax.experimental.pallas import tpu as pltpu
from jax.experimental.pallas import tpu_sc as plsc

B, C, H, W_IMG, K = 8, 768, 32, 32, 1024
HW = H * W_IMG
_NTILE = 16
CPT = C // _NTILE
_NCV = CPT // 16


def _norm_body(c_ref, o_ref):
    c = c_ref[...]
    n = jnp.sqrt(jnp.sum(c * c, axis=1, keepdims=True))
    o_ref[...] = (c / jnp.maximum(n, 1e-12)).astype(jnp.bfloat16)


def _assign_body(f_ref, cn_ref, idx_ref, cnt_ref):
    f = f_ref[0]
    n = jnp.sqrt(jnp.sum(f * f, axis=0, keepdims=True))
    fn = (f / jnp.maximum(n, 1e-12)).astype(jnp.bfloat16)
    cn = cn_ref[...]
    scores = lax.dot_general(cn, fn, (((1,), (0,)), ((), ())),
                             preferred_element_type=jnp.float32)
    m = jnp.max(scores, axis=0, keepdims=True)
    kio = lax.broadcasted_iota(jnp.int32, (K, HW), 0)
    idx = jnp.min(jnp.where(scores >= m, kio, K), axis=0)
    idx_ref[0, 0, :] = idx
    cnt_ref[0, 0, :] = jnp.sum((kio == idx[None, :]).astype(jnp.float32),
                               axis=1)


def _assign(f3, cn):
    return pl.pallas_call(
        _assign_body,
        grid=(B,),
        in_specs=[
            pl.BlockSpec((1, C, HW), lambda b: (b, 0, 0)),
            pl.BlockSpec((K, C), lambda b: (0, 0)),
        ],
        out_specs=[
            pl.BlockSpec((1, 1, HW), lambda b: (b, 0, 0)),
            pl.BlockSpec((1, 1, K), lambda b: (b, 0, 0)),
        ],
        out_shape=[
            jax.ShapeDtypeStruct((B, 1, HW), jnp.int32),
            jax.ShapeDtypeStruct((B, 1, K), jnp.float32),
        ],
    )(f3, cn)


def _sc_body(ft_hbm, idx_hbm, z_hbm, g_hbm, ftile, acc, idxv):
    cid = lax.axis_index("c")
    sid = lax.axis_index("s")
    col0 = sid * CPT
    iota16 = lax.iota(jnp.int32, 16)
    rows = [jv * 16 + iota16 for jv in range(_NCV)]
    zero16 = jnp.zeros((16,), jnp.int32)

    for bb in range(B // 2):
        b = cid * (B // 2) + bb

        pltpu.sync_copy(idx_hbm.at[pl.ds(b * HW, HW)], idxv)
        pltpu.sync_copy(ft_hbm.at[pl.ds(b * C + col0, CPT)], ftile)
        pltpu.sync_copy(z_hbm, acc)

        @plsc.parallel_loop(0, HW, 1, unroll=8)
        def _scat(p):
            pcol = zero16 + p
            s16 = plsc.load_gather(idxv, [pcol])
            for jv in range(_NCV):
                x16 = plsc.load_gather(ftile, [rows[jv], pcol])
                plsc.addupdate_scatter(acc, [rows[jv], s16], x16)

        pltpu.sync_copy(acc, g_hbm.at[pl.ds(b * C + col0, CPT)])


def _sc_scatter(ft_flat, idx_flat, zeros_ck):
    mesh = plsc.VectorSubcoreMesh(core_axis_name="c", subcore_axis_name="s")
    fn = functools.partial(
        pl.kernel,
        mesh=mesh,
        compiler_params=pltpu.CompilerParams(
            use_tc_tiling_on_sc=False, needs_layout_passes=False),
        out_type=jax.ShapeDtypeStruct((B * C, K), jnp.float32),
        scratch_types=[
            pltpu.VMEM((CPT, HW), jnp.float32),
            pltpu.VMEM((CPT, K), jnp.float32),
            pltpu.VMEM((HW,), jnp.int32),
        ],
    )(_sc_body)
    return fn(ft_flat, idx_flat, zeros_ck)


def _finish_body(f_ref, g_ref, cnt_ref, idx_ref, w_ref, b_ref, out_ref):
    f = f_ref[0]
    g = g_ref[0]
    cnt = cnt_ref[0]
    idx = idx_ref[0]
    cents = (g / jnp.maximum(cnt, 1.0)).astype(jnp.bfloat16)
    kio = lax.broadcasted_iota(jnp.int32, (K, HW), 0)
    mask = (kio == idx).astype(jnp.bfloat16)
    gath = lax.dot_general(cents, mask, (((1,), (0,)), ((), ())),
                           preferred_element_type=jnp.float32)
    delta = gath - f
    alpha = jnp.exp(-jnp.mean(delta * delta, axis=0, keepdims=True))
    x = f + alpha * delta
    out = lax.dot_general(w_ref[...], x, (((1,), (0,)), ((), ())),
                          preferred_element_type=jnp.float32)
    out_ref[0] = jnp.maximum(out + b_ref[...], 0.0)


def _finish(f3, g, cnt, idx3, Wfc, b2):
    return pl.pallas_call(
        _finish_body,
        grid=(B,),
        in_specs=[
            pl.BlockSpec((1, C, HW), lambda b: (b, 0, 0)),
            pl.BlockSpec((1, C, K), lambda b: (b, 0, 0)),
            pl.BlockSpec((1, 1, K), lambda b: (b, 0, 0)),
            pl.BlockSpec((1, 1, HW), lambda b: (b, 0, 0)),
            pl.BlockSpec((C, C), lambda b: (0, 0)),
            pl.BlockSpec((C, 1), lambda b: (0, 0)),
        ],
        out_specs=pl.BlockSpec((1, C, HW), lambda b: (b, 0, 0)),
        out_shape=jax.ShapeDtypeStruct((B, C, HW), jnp.float32),
    )(f3, g, cnt, idx3, Wfc, b2)


def kernel(features, centroids, Wfc, bfc):
    f3 = features.reshape(B, C, HW)
    cn = pl.pallas_call(
        _norm_body,
        out_shape=jax.ShapeDtypeStruct((K, C), jnp.bfloat16),
    )(centroids)
    idx3, cnt = _assign(f3, cn)
    zeros_ck = jnp.zeros((CPT, K), jnp.float32)
    g = _sc_scatter(features.reshape(B * C, HW), idx3.reshape(B * HW),
                    zeros_ck)
    out = _finish(f3, g.reshape(B, C, K), cnt, idx3, Wfc,
                  bfc.reshape(C, 1))
    return out.reshape(B, C, H, W_IMG)
